# baseline (speedup 1.0000x reference)
import functools

import jax
import jax.numpy as jnp
from jax import lax
from jax.experimental import pallas as pl
from jax.experimental.pallas import tpu as pltpu
from jax.experimental.pallas import tpu_sc as plsc

_B = 1024
_RANK = 32
_N_ENT = 100000
_LANES = 16
_NC = 2
_NS = 16


def _sc_gather_call(entity_emb, rel_emb, heads, rels, tails):
    nw = _NC * _NS
    bpw = _B // nw
    mesh = plsc.VectorSubcoreMesh(core_axis_name="c", subcore_axis_name="s")

    @functools.partial(
        pl.kernel,
        mesh=mesh,
        out_type=(
            jax.ShapeDtypeStruct((_B, _RANK), jnp.float32),
            jax.ShapeDtypeStruct((_B, _RANK), jnp.float32),
        ),
        scratch_types=[
            pltpu.VMEM((bpw,), jnp.int32),
            pltpu.VMEM((bpw,), jnp.int32),
            pltpu.VMEM((bpw,), jnp.int32),
            pltpu.VMEM((bpw, _RANK), jnp.float32),
            pltpu.VMEM((bpw, _RANK), jnp.float32),
            pltpu.VMEM((bpw, _RANK), jnp.float32),
            pltpu.SemaphoreType.DMA,
            pltpu.SemaphoreType.DMA,
            pltpu.SemaphoreType.DMA,
        ],
        compiler_params=pltpu.CompilerParams(use_tc_tiling_on_sc=False),
    )
    def k(ent_hbm, rel_hbm, h_hbm, r_hbm, t_hbm, q_out, te_out,
          hv, rv, tv, lhs_v, rel_v, te_v, sem1, sem2, sem3):
        wid = lax.axis_index("s") * _NC + lax.axis_index("c")
        base = wid * bpw
        pltpu.sync_copy(h_hbm.at[pl.ds(base, bpw)], hv)
        pltpu.sync_copy(r_hbm.at[pl.ds(base, bpw)], rv)
        pltpu.sync_copy(t_hbm.at[pl.ds(base, bpw)], tv)
        c1 = pltpu.async_copy(ent_hbm.at[hv], lhs_v, sem1)
        c2 = pltpu.async_copy(rel_hbm.at[rv], rel_v, sem2)
        c3 = pltpu.async_copy(ent_hbm.at[tv], te_v, sem3)
        c1.wait()
        c2.wait()
        for i in range(bpw):
            for j in range(_RANK // _LANES):
                sl = pl.ds(j * _LANES, _LANES)
                lhs_v[i, sl] = lhs_v[i, sl] * rel_v[i, sl]
        pltpu.sync_copy(lhs_v, q_out.at[pl.ds(base, bpw)])
        c3.wait()
        pltpu.sync_copy(te_v, te_out.at[pl.ds(base, bpw)])

    return k(entity_emb, rel_emb, heads, rels, tails)


_TILE_B = 32


def _tc_body(q_ref, te_ref, tgt_ref, embt_ref, masked_ref, ranks_ref):
    ts = jnp.sum(q_ref[...] * te_ref[...], axis=1, keepdims=True)
    scores = jnp.dot(q_ref[...], embt_ref[...],
                     preferred_element_type=jnp.float32)
    cols = lax.broadcasted_iota(jnp.int32, (_TILE_B, _N_ENT), 1)
    masked = jnp.where(cols == tgt_ref[...], -1000000.0, scores)
    masked_ref[...] = masked
    ranks_ref[...] = 1.0 + jnp.sum(
        (masked >= ts).astype(jnp.float32), axis=1, keepdims=True)


def _tc_score_call(q, tgt_e, tgt, embt):
    return pl.pallas_call(
        _tc_body,
        grid=(_B // _TILE_B,),
        in_specs=[
            pl.BlockSpec((_TILE_B, _RANK), lambda i: (i, 0)),
            pl.BlockSpec((_TILE_B, _RANK), lambda i: (i, 0)),
            pl.BlockSpec((_TILE_B, 1), lambda i: (i, 0)),
            pl.BlockSpec((_RANK, _N_ENT), lambda i: (0, 0)),
        ],
        out_specs=[
            pl.BlockSpec((_TILE_B, _N_ENT), lambda i: (i, 0)),
            pl.BlockSpec((_TILE_B, 1), lambda i: (i, 0)),
        ],
        out_shape=[
            jax.ShapeDtypeStruct((_B, _N_ENT), jnp.float32),
            jax.ShapeDtypeStruct((_B, 1), jnp.float32),
        ],
        compiler_params=pltpu.CompilerParams(
            dimension_semantics=("parallel",)),
    )(q, tgt_e, tgt, embt)


def kernel(queries, entity_emb, rel_emb):
    heads = queries[:, 0].astype(jnp.int32)
    rels = queries[:, 1].astype(jnp.int32)
    tails = queries[:, 2].astype(jnp.int32)
    q, tgt_e = _sc_gather_call(entity_emb, rel_emb, heads, rels, tails)
    embt = entity_emb.T
    masked, ranks = _tc_score_call(q, tgt_e, tails[:, None], embt)
    return ranks.reshape(_B), masked


# speedup vs baseline: 1.0027x; 1.0027x over previous
import functools

import jax
import jax.numpy as jnp
from jax import lax
from jax.experimental import pallas as pl
from jax.experimental.pallas import tpu as pltpu
from jax.experimental.pallas import tpu_sc as plsc

_B = 1024
_RANK = 32
_N_ENT = 100000
_LANES = 16
_NC = 2
_NS = 16


def _sc_gather_call(entity_emb, rel_emb, heads, rels, tails):
    nw = _NC * _NS
    bpw = _B // nw
    mesh = plsc.VectorSubcoreMesh(core_axis_name="c", subcore_axis_name="s")

    @functools.partial(
        pl.kernel,
        mesh=mesh,
        out_type=(
            jax.ShapeDtypeStruct((_B, _RANK), jnp.float32),
            jax.ShapeDtypeStruct((_B, _RANK), jnp.float32),
        ),
        scratch_types=[
            pltpu.VMEM((bpw,), jnp.int32),
            pltpu.VMEM((bpw,), jnp.int32),
            pltpu.VMEM((bpw,), jnp.int32),
            pltpu.VMEM((bpw, _RANK), jnp.float32),
            pltpu.VMEM((bpw, _RANK), jnp.float32),
            pltpu.VMEM((bpw, _RANK), jnp.float32),
            pltpu.SemaphoreType.DMA,
            pltpu.SemaphoreType.DMA,
            pltpu.SemaphoreType.DMA,
        ],
        compiler_params=pltpu.CompilerParams(use_tc_tiling_on_sc=False),
    )
    def k(ent_hbm, rel_hbm, h_hbm, r_hbm, t_hbm, q_out, te_out,
          hv, rv, tv, lhs_v, rel_v, te_v, sem1, sem2, sem3):
        wid = lax.axis_index("s") * _NC + lax.axis_index("c")
        base = wid * bpw
        pltpu.sync_copy(h_hbm.at[pl.ds(base, bpw)], hv)
        pltpu.sync_copy(r_hbm.at[pl.ds(base, bpw)], rv)
        pltpu.sync_copy(t_hbm.at[pl.ds(base, bpw)], tv)
        c1 = pltpu.async_copy(ent_hbm.at[hv], lhs_v, sem1)
        c2 = pltpu.async_copy(rel_hbm.at[rv], rel_v, sem2)
        c3 = pltpu.async_copy(ent_hbm.at[tv], te_v, sem3)
        c1.wait()
        c2.wait()
        for i in range(bpw):
            for j in range(_RANK // _LANES):
                sl = pl.ds(j * _LANES, _LANES)
                lhs_v[i, sl] = lhs_v[i, sl] * rel_v[i, sl]
        pltpu.sync_copy(lhs_v, q_out.at[pl.ds(base, bpw)])
        c3.wait()
        pltpu.sync_copy(te_v, te_out.at[pl.ds(base, bpw)])

    return k(entity_emb, rel_emb, heads, rels, tails)


_TILE_B = 32


def _tc_body(q_ref, te_ref, tgt_ref, embt_ref, masked_ref, ranks_ref):
    ts = jnp.sum(q_ref[...] * te_ref[...], axis=1, keepdims=True)
    scores = jnp.dot(q_ref[...], embt_ref[...],
                     preferred_element_type=jnp.float32)
    cols = lax.broadcasted_iota(jnp.int32, (_TILE_B, _N_ENT), 1)
    masked = jnp.where(cols == tgt_ref[...], -1000000.0, scores)
    masked_ref[...] = masked
    ranks_ref[...] = 1.0 + jnp.sum(
        (masked >= ts).astype(jnp.float32), axis=1, keepdims=True)


def _tc_score_call(q, tgt_e, tgt, embt):
    return pl.pallas_call(
        _tc_body,
        grid=(_B // _TILE_B,),
        in_specs=[
            pl.BlockSpec((_TILE_B, _RANK), lambda i: (i, 0)),
            pl.BlockSpec((_TILE_B, _RANK), lambda i: (i, 0)),
            pl.BlockSpec((_TILE_B, 1), lambda i: (i, 0)),
            pl.BlockSpec((_RANK, _N_ENT), lambda i: (0, 0)),
        ],
        out_specs=[
            pl.BlockSpec((_TILE_B, _N_ENT), lambda i: (i, 0)),
            pl.BlockSpec((_TILE_B, 1), lambda i: (i, 0)),
        ],
        out_shape=[
            jax.ShapeDtypeStruct((_B, _N_ENT), jnp.float32),
            jax.ShapeDtypeStruct((_B, 1), jnp.float32),
        ],
        compiler_params=pltpu.CompilerParams(
            dimension_semantics=("parallel",),
            allow_input_fusion=(False, False, False, True)),
    )(q, tgt_e, tgt, embt)


def kernel(queries, entity_emb, rel_emb):
    heads = queries[:, 0].astype(jnp.int32)
    rels = queries[:, 1].astype(jnp.int32)
    tails = queries[:, 2].astype(jnp.int32)
    q, tgt_e = _sc_gather_call(entity_emb, rel_emb, heads, rels, tails)
    embt = entity_emb.T
    masked, ranks = _tc_score_call(q, tgt_e, tails[:, None], embt)
    return ranks.reshape(_B), masked
